# baseline (speedup 1.0000x reference)
import functools

import jax
import jax.numpy as jnp
from jax import lax
from jax.experimental import pallas as pl
from jax.experimental.pallas import tpu as pltpu
from jax.experimental.pallas import tpu_sc as plsc

N_TASK, N_PE, N_ROUTER = 50000, 18432, 18432
H = 128

NP_TASK = 51200
NP_PR = 20480

NP_MAX = 50176
NWR = 784
MSEG = 2048
G = 128
GM = 64


def _pad_edges(e_src, e_dst, mult, trash):
    e = e_src.shape[0]
    ep = ((e + mult - 1) // mult) * mult
    if ep != e:
        e_src = jnp.pad(e_src, (0, ep - e), constant_values=0)
        e_dst = jnp.pad(e_dst, (0, ep - e), constant_values=trash)
    return e_src, e_dst


def _seg_sum(x, e_src, e_dst, n_dst, n_dst_pad, n_pass):
    e_src, e_dst = _pad_edges(e_src, e_dst, 512, n_dst)
    ep = e_src.shape[0]
    per_w = ep // 32
    nvr = per_w // 16
    cap = per_w + G + 16
    R = n_dst_pad // n_pass
    r16 = R // 16
    chunk = 80
    nch = r16 // chunk

    mesh = plsc.VectorSubcoreMesh(core_axis_name="c", subcore_axis_name="s")

    @functools.partial(
        pl.kernel,
        mesh=mesh,
        out_type=jax.ShapeDtypeStruct((2, n_dst_pad, H), jnp.float32),
        scratch_types=[
            pltpu.VMEM((per_w,), jnp.int32),
            pltpu.VMEM((per_w,), jnp.int32),
            pltpu.VMEM((cap,), jnp.int32),
            pltpu.VMEM((cap,), jnp.int32),
            pltpu.VMEM((G,), jnp.int32),
            pltpu.VMEM((G,), jnp.int32),
            pltpu.VMEM((G, H), jnp.float32),
            pltpu.VMEM_SHARED((R + 16, H), jnp.float32),
            pltpu.SemaphoreType.DMA,
        ],
    )
    def k(es, ed, xr, out, sbuf, dbuf, cs, cd, gidx, ldst, rows, acc, sem):
        c = lax.axis_index("c")
        s = lax.axis_index("s")
        gw = c * 16 + s
        base = gw * per_w
        pltpu.sync_copy(es.at[pl.ds(base, per_w)], sbuf)
        pltpu.sync_copy(ed.at[pl.ds(base, per_w)], dbuf)
        dummy = jnp.full((16,), R, jnp.int32)
        zero16 = jnp.zeros((16,), jnp.int32)

        for p in range(n_pass):
            lo = p * R
            def zrow(r, _):
                for j in range(H // 16):
                    rows[r, pl.ds(16 * j, 16)] = jnp.zeros((16,), jnp.float32)
                return 0
            lax.fori_loop(0, chunk, zrow, 0)

            def zacc(kk, _):
                off = s * r16 + kk * chunk
                pltpu.sync_copy(rows.at[pl.ds(0, chunk)],
                                acc.at[pl.ds(off, chunk)])
                return 0
            lax.fori_loop(0, nch, zacc, 0)
            pltpu.sync_copy(rows.at[pl.ds(0, 16)], acc.at[pl.ds(R, 16)])
            plsc.subcore_barrier()

            def scan(i, _):
                dvec = dbuf[pl.ds(i * 16, 16)]
                svec = sbuf[pl.ds(i * 16, 16)]
                m = (dvec >= lo) & (dvec < lo + R)
                cd[pl.ds(i * 16, 16)] = jnp.where(m, dvec - lo, R)
                cs[pl.ds(i * 16, 16)] = jnp.where(m, svec, 0)
                return 0
            lax.fori_loop(0, nvr, scan, 0)
            cur = per_w
            for kk in range(G // 16):
                cd[pl.ds(cur + kk * 16, 16)] = dummy
                cs[pl.ds(cur + kk * 16, 16)] = zero16
            nbg = (cur + G - 1) // G

            def gb(b, _):
                pltpu.async_copy(xr.at[cs.at[pl.ds(b * G, G)]], rows,
                                 sem).wait()
                pltpu.sync_copy(rows, acc.at[cd.at[pl.ds(b * G, G)]],
                                add=True)
                return 0
            lax.fori_loop(0, nbg, gb, 0)
            plsc.subcore_barrier()

            def wout(kk, _):
                off = s * r16 + kk * chunk
                pltpu.sync_copy(acc.at[pl.ds(off, chunk)],
                                out.at[c, pl.ds(lo + off, chunk)])
                return 0
            lax.fori_loop(0, nch, wout, 0)
            plsc.subcore_barrier()

    return k(e_src, e_dst, x)


def _seg_max(x, e_src, e_dst):
    e_src, e_dst = _pad_edges(e_src, e_dst, MSEG, N_TASK)
    ep = e_src.shape[0]
    nseg = ep // MSEG
    nvr = MSEG // 16
    cap = MSEG + GM + 16

    mesh = plsc.VectorSubcoreMesh(core_axis_name="c", subcore_axis_name="s")

    @functools.partial(
        pl.kernel,
        mesh=mesh,
        out_type=jax.ShapeDtypeStruct((2, NP_MAX, 64), jnp.float32),
        scratch_types=[
            pltpu.VMEM((MSEG,), jnp.int32),
            pltpu.VMEM((MSEG,), jnp.int32),
            pltpu.VMEM((cap,), jnp.int32),
            pltpu.VMEM((cap,), jnp.int32),
            pltpu.VMEM((GM,), jnp.int32),
            pltpu.VMEM((GM, H), jnp.float32),
            pltpu.VMEM((NWR + 1, 64), jnp.float32),
            pltpu.SemaphoreType.DMA,
        ],
    )
    def k(es, ed, xr, out, sbuf, dbuf, cs, cd, gidx, rows, acc, sem):
        c = lax.axis_index("c")
        s = lax.axis_index("s")
        neg = jnp.full((16,), -jnp.inf, jnp.float32)
        dummy = jnp.full((16,), NWR, jnp.int32)
        zero16 = jnp.zeros((16,), jnp.int32)

        for t in range(4):
            rng = t * 16 + s
            lo = rng * NWR

            def init(r, _):
                for j in range(4):
                    acc[r, pl.ds(16 * j, 16)] = neg
                return 0
            lax.fori_loop(0, NWR + 1, init, 0)

            def seg(sg, _):
                pltpu.sync_copy(es.at[pl.ds(sg * MSEG, MSEG)], sbuf)
                pltpu.sync_copy(ed.at[pl.ds(sg * MSEG, MSEG)], dbuf)

                def scan(i, _):
                    dvec = dbuf[pl.ds(i * 16, 16)]
                    svec = sbuf[pl.ds(i * 16, 16)]
                    m = (dvec >= lo) & (dvec < lo + NWR)
                    cd[pl.ds(i * 16, 16)] = jnp.where(m, dvec - lo, NWR)
                    cs[pl.ds(i * 16, 16)] = jnp.where(m, svec, 0)
                    return 0
                lax.fori_loop(0, nvr, scan, 0)
                cur = MSEG
                for kk in range(GM // 16):
                    cd[pl.ds(cur + kk * 16, 16)] = dummy
                    cs[pl.ds(cur + kk * 16, 16)] = zero16
                nbg = (cur + GM - 1) // GM

                def gb(b, _):
                    pltpu.async_copy(xr.at[cs.at[pl.ds(b * GM, GM)]], rows,
                                     sem).wait()

                    def acc_e(e, _):
                        d = cd[pl.ds(b * GM + e, 16)][0]
                        for j in range(4):
                            sl = pl.ds(16 * j, 16)
                            sr = pl.ds(c * 64 + 16 * j, 16)
                            acc[d, sl] = jnp.maximum(acc[d, sl], rows[e, sr])
                        return 0
                    lax.fori_loop(0, GM, acc_e, 0)
                    return 0
                lax.fori_loop(0, nbg, gb, 0)
                return 0
            lax.fori_loop(0, nseg, seg, 0)

            pltpu.sync_copy(acc.at[pl.ds(0, NWR)],
                            out.at[c, pl.ds(lo, NWR)])

    return k(e_src, e_dst, x)


def _row_block(nrows, wfc):
    return pl.BlockSpec((nrows, wfc), lambda i: (i, 0))


def _pair_block(nrows, wfc):
    return pl.BlockSpec((2, nrows, wfc), lambda i: (0, i, 0))


def _full_block(shape):
    return pl.BlockSpec(shape, lambda i: tuple(0 for _ in shape))


def _emb_transform(pe_emb, rt_emb, w_pet, w_per, w_rr, w_rpe):
    def body(pe_r, rt_r, wa, wb, wc, wd, o0, o1, o2, o3):
        o0[...] = jnp.dot(pe_r[...], wa[...],
                          preferred_element_type=jnp.float32)
        o1[...] = jnp.dot(pe_r[...], wb[...],
                          preferred_element_type=jnp.float32)
        o2[...] = jnp.dot(rt_r[...], wc[...],
                          preferred_element_type=jnp.float32)
        o3[...] = jnp.dot(rt_r[...], wd[...],
                          preferred_element_type=jnp.float32)

    return pl.pallas_call(
        body,
        grid=(1,),
        in_specs=[_full_block((9, 16))] * 2 + [_full_block((16, H))] * 4,
        out_specs=[_full_block((9, H))] * 4,
        out_shape=[jax.ShapeDtypeStruct((9, H), jnp.float32)] * 4,
    )(pe_emb, rt_emb, w_pet, w_per, w_rr, w_rpe)


def _dense_task(amax, asum, apet, x, wg, wq, wp, wr, b, wo=None, bo=None):
    nb, br = 25, 2000
    n_extra = 0 if wp is None else 1
    n_out = 2 if wo is not None else H

    def body(*refs):
        am, as_, ap, x_r, wg_r, wq_r = refs[:6]
        idx = 6
        if wp is not None:
            wp_r = refs[idx]
            idx += 1
        wr_r, b_r = refs[idx], refs[idx + 1]
        idx += 2
        if wo is not None:
            wo_r, bo_r = refs[idx], refs[idx + 1]
            idx += 2
        out = refs[idx]
        m0 = am[0]
        m1 = am[1]
        acc = jnp.dot(jnp.where(jnp.isfinite(m0), m0, 0.0),
                      wg_r[pl.ds(0, 64), :],
                      preferred_element_type=jnp.float32)
        acc += jnp.dot(jnp.where(jnp.isfinite(m1), m1, 0.0),
                       wg_r[pl.ds(64, 64), :],
                       preferred_element_type=jnp.float32)
        acc += jnp.dot(as_[0] + as_[1], wq_r[...],
                       preferred_element_type=jnp.float32)
        ap_s = ap[0] + ap[1]
        if wp is not None:
            acc += jnp.dot(ap_s, wp_r[...],
                           preferred_element_type=jnp.float32)
        else:
            acc += ap_s
        acc += jnp.dot(x_r[...], wr_r[...],
                       preferred_element_type=jnp.float32)
        act = jax.nn.relu((acc + b_r[...]) / 3.0)
        if wo is not None:
            out[...] = jnp.dot(act, wo_r[...],
                               preferred_element_type=jnp.float32) + bo_r[...]
        else:
            out[...] = act

    in_specs = [_pair_block(br, 64), _pair_block(br, H), _pair_block(br, H),
                _row_block(br, H), _full_block((H, H)), _full_block((H, H))]
    args = [amax, asum, apet, x, wg, wq]
    if wp is not None:
        in_specs.append(_full_block((H, H)))
        args.append(wp)
    in_specs += [_full_block((H, H)), _full_block((1, H))]
    args += [wr, b]
    if wo is not None:
        in_specs += [_full_block((H, 2)), _full_block((1, 2))]
        args += [wo, bo]

    return pl.pallas_call(
        body,
        grid=(nb,),
        in_specs=in_specs,
        out_specs=_row_block(br, n_out),
        out_shape=jax.ShapeDtypeStruct((N_TASK, n_out), jnp.float32),
    )(*args)


def _dense_pr(a1, a2, xe, w1, w2, wr, b):
    nb, br = 9, 2048
    we = xe.shape[1]

    def body(*refs):
        s1, s2, xe_r = refs[0], refs[1], refs[2]
        idx = 3
        if w1 is not None:
            w1_r = refs[idx]
            idx += 1
        if w2 is not None:
            w2_r = refs[idx]
            idx += 1
        wr_r, b_r, out = refs[idx], refs[idx + 1], refs[idx + 2]
        v1 = s1[0] + s1[1]
        if w1 is not None:
            acc = jnp.dot(v1, w1_r[...], preferred_element_type=jnp.float32)
        else:
            acc = v1
        v2 = s2[0] + s2[1]
        if w2 is not None:
            acc += jnp.dot(v2, w2_r[...], preferred_element_type=jnp.float32)
        else:
            acc += v2
        acc += jnp.dot(xe_r[...], wr_r[...],
                       preferred_element_type=jnp.float32)
        out[...] = jax.nn.relu((acc + b_r[...]) / 2.0)

    in_specs = [_pair_block(br, H), _pair_block(br, H), _row_block(br, we)]
    args = [a1, a2, xe]
    if w1 is not None:
        in_specs.append(_full_block((H, H)))
        args.append(w1)
    if w2 is not None:
        in_specs.append(_full_block((H, H)))
        args.append(w2)
    in_specs += [_full_block((we, H)), _full_block((1, H))]
    args += [wr, b]

    return pl.pallas_call(
        body,
        grid=(nb,),
        in_specs=in_specs,
        out_specs=_row_block(br, H),
        out_shape=jax.ShapeDtypeStruct((N_PE, H), jnp.float32),
    )(*args)


def kernel(x_task, x_pe, x_router, ei_tt_gen, ei_tt_req, ei_t_pe, ei_pe_t,
           ei_rr, ei_r_pe, ei_pe_r, params):
    bs = x_pe.shape[0] // 9
    xpe0 = jnp.tile(params["pe_emb"], (bs, 1))
    xrt0 = jnp.tile(params["router_emb"], (bs, 1))

    l0, l1 = params["layer0"], params["layer1"]

    t_pet, t_per, t_rr, t_rpe = _emb_transform(
        params["pe_emb"], params["router_emb"], l0["pe_t"]["Wrel"],
        l0["pe_r"]["Wrel"], l0["rr"]["Wrel"], l0["r_pe"]["Wrel"])
    x_pet = jnp.tile(t_pet, (bs, 1))
    x_per = jnp.tile(t_per, (bs, 1))
    x_rr = jnp.tile(t_rr, (bs, 1))
    x_rpe = jnp.tile(t_rpe, (bs, 1))

    a_max = _seg_max(x_task, ei_tt_gen[0], ei_tt_gen[1])
    a_req = _seg_sum(x_task, ei_tt_req[0], ei_tt_req[1], N_TASK, NP_TASK, 5)
    a_tpe = _seg_sum(x_task, ei_t_pe[0], ei_t_pe[1], N_PE, NP_PR, 2)
    a_pet = _seg_sum(x_pet, ei_pe_t[0], ei_pe_t[1], N_TASK, NP_TASK, 5)
    a_rr = _seg_sum(x_rr, ei_rr[0], ei_rr[1], N_ROUTER, NP_PR, 2)
    a_rpe = _seg_sum(x_rpe, ei_r_pe[0], ei_r_pe[1], N_PE, NP_PR, 2)
    a_per = _seg_sum(x_per, ei_pe_r[0], ei_pe_r[1], N_ROUTER, NP_PR, 2)

    b_t0 = (l0["tt_gen"]["b"] + l0["tt_req"]["b"]
            + l0["pe_t"]["b"]).reshape(1, H)
    wr_t0 = l0["tt_gen"]["Wroot"] + l0["tt_req"]["Wroot"] + l0["pe_t"]["Wroot"]
    t1 = _dense_task(a_max, a_req, a_pet, x_task, l0["tt_gen"]["Wrel"],
                     l0["tt_req"]["Wrel"], None, wr_t0, b_t0)

    b_p0 = (l0["t_pe"]["b"] + l0["r_pe"]["b"]).reshape(1, H)
    wr_p0 = l0["t_pe"]["Wroot"] + l0["r_pe"]["Wroot"]
    p1 = _dense_pr(a_tpe, a_rpe, xpe0, l0["t_pe"]["Wrel"], None, wr_p0, b_p0)

    b_r0 = (l0["rr"]["b"] + l0["pe_r"]["b"]).reshape(1, H)
    wr_r0 = l0["rr"]["Wroot"] + l0["pe_r"]["Wroot"]
    r1 = _dense_pr(a_rr, a_per, xrt0, None, None, wr_r0, b_r0)

    b_max = _seg_max(t1, ei_tt_gen[0], ei_tt_gen[1])
    b_req = _seg_sum(t1, ei_tt_req[0], ei_tt_req[1], N_TASK, NP_TASK, 5)
    b_tpe = _seg_sum(t1, ei_t_pe[0], ei_t_pe[1], N_PE, NP_PR, 2)
    b_pet = _seg_sum(p1, ei_pe_t[0], ei_pe_t[1], N_TASK, NP_TASK, 5)
    b_rr = _seg_sum(r1, ei_rr[0], ei_rr[1], N_ROUTER, NP_PR, 2)
    b_rpe = _seg_sum(r1, ei_r_pe[0], ei_r_pe[1], N_PE, NP_PR, 2)
    b_per = _seg_sum(p1, ei_pe_r[0], ei_pe_r[1], N_ROUTER, NP_PR, 2)

    b_t1 = (l1["tt_gen"]["b"] + l1["tt_req"]["b"]
            + l1["pe_t"]["b"]).reshape(1, H)
    wr_t1 = l1["tt_gen"]["Wroot"] + l1["tt_req"]["Wroot"] + l1["pe_t"]["Wroot"]
    task_out = _dense_task(
        b_max, b_req, b_pet, t1, l1["tt_gen"]["Wrel"], l1["tt_req"]["Wrel"],
        l1["pe_t"]["Wrel"], wr_t1, b_t1, params["W_out"],
        params["b_out"].reshape(1, 2))

    b_p1 = (l1["t_pe"]["b"] + l1["r_pe"]["b"]).reshape(1, H)
    wr_p1 = l1["t_pe"]["Wroot"] + l1["r_pe"]["Wroot"]
    pe2 = _dense_pr(b_tpe, b_rpe, p1, l1["t_pe"]["Wrel"],
                    l1["r_pe"]["Wrel"], wr_p1, b_p1)

    b_r1 = (l1["rr"]["b"] + l1["pe_r"]["b"]).reshape(1, H)
    wr_r1 = l1["rr"]["Wroot"] + l1["pe_r"]["Wroot"]
    rt2 = _dense_pr(b_rr, b_per, r1, l1["rr"]["Wrel"],
                    l1["pe_r"]["Wrel"], wr_r1, b_r1)

    return task_out, pe2, rt2
